# Initial kernel scaffold; baseline (speedup 1.0000x reference)
import jax
import jax.numpy as jnp
from jax.experimental import pallas as pl


def kernel(x, edge_index, edge_attr, batch, W1, b1, W2, b2, W_root, gamma, beta):
    raise NotImplementedError("write your pallas kernel here")


# speedup vs baseline: 1.9619x; 1.9619x over previous
import functools

import jax
import jax.numpy as jnp
from jax import lax
from jax.experimental import pallas as pl
from jax.experimental.pallas import tpu as pltpu
from jax.experimental.pallas import tpu_sc as plsc

_NC = 2
_NS = 16
_LANES = 16

_K = 128
_DEGW = 16


def _linear_body(xb, wc, b1r, a0, a1, b0, b1o, xr):
    y = jnp.dot(xb[...], wc[...], preferred_element_type=jnp.float32)
    a0[...] = y[:, 0:128] + b1r[:, 0:128]
    a1[...] = y[:, 128:256] + b1r[:, 128:256]
    b0[...] = y[:, 256:384]
    b1o[...] = y[:, 384:512]
    xr[...] = y[:, 512:768]


def _node_linear(x_pad, wcat, b1, npad, rb):
    grid = npad // rb
    f32 = jnp.float32
    outs = [
        jax.ShapeDtypeStruct((npad, 128), f32),
        jax.ShapeDtypeStruct((npad, 128), f32),
        jax.ShapeDtypeStruct((npad, 128), f32),
        jax.ShapeDtypeStruct((npad, 128), f32),
        jax.ShapeDtypeStruct((npad, 256), f32),
    ]
    half_spec = pl.BlockSpec((rb, 128), lambda i: (i, 0))
    full_spec = pl.BlockSpec((rb, 256), lambda i: (i, 0))
    return pl.pallas_call(
        _linear_body,
        grid=(grid,),
        in_specs=[
            pl.BlockSpec((rb, 256), lambda i: (i, 0)),
            pl.BlockSpec((256, 768), lambda i: (0, 0)),
            pl.BlockSpec((1, 256), lambda i: (0, 0)),
        ],
        out_specs=[half_spec, half_spec, half_spec, half_spec, full_spec],
        out_shape=outs,
    )(x_pad, wcat, b1.reshape(1, 256))


def _sc_edge_kernel(a0, a1, b0, b1, dstp, srcp, z, npad, ep):
    ec = ep // (_NC * _NS)
    nb = ec // _K
    rpt = npad // _NS
    f32 = jnp.float32

    def body(a0_h, a1_h, b0_h, b1_h, dst_h, src_h, z_h,
             s_out,
             dstw, srcw, bufa, bufb, s_sh,
             sema, semb):
        cid = lax.axis_index("c")
        sid = lax.axis_index("s")
        gid = cid * _NS + sid
        base = gid * ec

        for h in range(3):
            ah = a0_h if h == 0 else a1_h
            bh = b0_h if h == 0 else b1_h
            pltpu.sync_copy(z_h, s_sh.at[pl.ds(sid * rpt, rpt)])
            plsc.subcore_barrier()

            if h == 2:
                def fill_ones(r, c):
                    for cc in range(128 // _LANES):
                        bufa[r, pl.ds(cc * _LANES, _LANES)] = jnp.full(
                            (_LANES,), 1.0, f32)
                    return c
                lax.fori_loop(0, _K, fill_ones, 0)

            def blk(b, c):
                off = base + b * _K
                pltpu.sync_copy(dst_h.at[pl.ds(off, _K)], dstw)
                if h < 2:
                    pltpu.sync_copy(src_h.at[pl.ds(off, _K)], srcw)
                    ga = pltpu.async_copy(ah.at[dstw], bufa, sema)
                    gb = pltpu.async_copy(bh.at[srcw], bufb, semb)
                    ga.wait()
                    gb.wait()

                    def row(r, c2):
                        for cc in range(128 // _LANES):
                            av = bufa[r, pl.ds(cc * _LANES, _LANES)]
                            bv = bufb[r, pl.ds(cc * _LANES, _LANES)]
                            bufa[r, pl.ds(cc * _LANES, _LANES)] = jnp.maximum(
                                av + bv, 0.0)
                        return c2
                    lax.fori_loop(0, _K, row, 0)

                pltpu.sync_copy(bufa, s_sh.at[dstw], add=True)
                return c
            lax.fori_loop(0, nb, blk, 0)
            plsc.subcore_barrier()
            pltpu.sync_copy(s_sh.at[pl.ds(sid * rpt, rpt)],
                            s_out.at[cid, h, pl.ds(sid * rpt, rpt)])
            plsc.subcore_barrier()

    mesh = plsc.VectorSubcoreMesh(core_axis_name="c", subcore_axis_name="s")
    kern = pl.kernel(
        body,
        out_type=jax.ShapeDtypeStruct((_NC, 3, npad, 128), f32),
        mesh=mesh,
        scratch_types=[
            pltpu.VMEM((_K,), jnp.int32),
            pltpu.VMEM((_K,), jnp.int32),
            pltpu.VMEM((_K, 128), f32),
            pltpu.VMEM((_K, 128), f32),
            pltpu.VMEM_SHARED((npad, 128), f32),
            pltpu.SemaphoreType.DMA,
            pltpu.SemaphoreType.DMA,
        ],
    )
    return kern(a0, a1, b0, b1, dstp, srcp, z)


def _agg_body(nblk, s_ref, xr_ref, w2_ref, b2r, p_ref, stats_ref,
              acc_ref):
    i = pl.program_id(0)
    sh0 = s_ref[0, 0] + s_ref[1, 0]
    sh1 = s_ref[0, 1] + s_ref[1, 1]
    p = jnp.dot(sh0, w2_ref[0:128, :], preferred_element_type=jnp.float32)
    p += jnp.dot(sh1, w2_ref[128:256, :], preferred_element_type=jnp.float32)
    dcol = s_ref[0, 2, :, 0:1] + s_ref[1, 2, :, 0:1]
    p = p + dcol * b2r[...] + xr_ref[...]
    p_ref[...] = p

    @pl.when(i == 0)
    def _init():
        acc_ref[...] = jnp.zeros_like(acc_ref)

    acc_ref[0:1, :] += jnp.sum(p, axis=0, keepdims=True)
    acc_ref[1:2, :] += jnp.sum(p * p, axis=0, keepdims=True)
    stats_ref[...] = acc_ref[...]


def _epilogue_a(s_out, xr, w2, b2, n, npad, rb):
    grid = n // rb
    f32 = jnp.float32
    return pl.pallas_call(
        functools.partial(_agg_body, grid),
        grid=(grid,),
        in_specs=[
            pl.BlockSpec((_NC, 3, rb, 128), lambda i: (0, 0, i, 0)),
            pl.BlockSpec((rb, 256), lambda i: (i, 0)),
            pl.BlockSpec((256, 256), lambda i: (0, 0)),
            pl.BlockSpec((1, 256), lambda i: (0, 0)),
        ],
        out_specs=[
            pl.BlockSpec((rb, 256), lambda i: (i, 0)),
            pl.BlockSpec((8, 256), lambda i: (0, 0)),
        ],
        out_shape=[
            jax.ShapeDtypeStruct((n, 256), f32),
            jax.ShapeDtypeStruct((8, 256), f32),
        ],
        scratch_shapes=[pltpu.VMEM((8, 256), f32)],
    )(s_out, xr, w2, b2.reshape(1, 256))


def _bn_body(n, p_ref, stats_ref, g_ref, be_ref, o_ref):
    inv_n = 1.0 / n
    mean = stats_ref[0:1, :] * inv_n
    ex2 = stats_ref[1:2, :] * inv_n
    var = ex2 - mean * mean
    rstd = lax.rsqrt(var + 1e-5)
    o_ref[...] = jnp.maximum(
        (p_ref[...] - mean) * rstd * g_ref[...] + be_ref[...], 0.0)


def _epilogue_b(p, stats, gamma, beta, n, rb):
    return pl.pallas_call(
        functools.partial(_bn_body, float(n)),
        grid=(n // rb,),
        in_specs=[
            pl.BlockSpec((rb, 256), lambda i: (i, 0)),
            pl.BlockSpec((8, 256), lambda i: (0, 0)),
            pl.BlockSpec((1, 256), lambda i: (0, 0)),
            pl.BlockSpec((1, 256), lambda i: (0, 0)),
        ],
        out_specs=pl.BlockSpec((rb, 256), lambda i: (i, 0)),
        out_shape=jax.ShapeDtypeStruct((n, 256), jnp.float32),
    )(p, stats, gamma.reshape(1, 256), beta.reshape(1, 256))


def kernel(x, edge_index, edge_attr, batch, W1, b1, W2, b2, W_root, gamma,
           beta):
    n, c = x.shape
    e = edge_index.shape[1]
    npad = 10240
    ep = ((e + (32 * _K) - 1) // (32 * _K)) * (32 * _K)
    dummy = n

    src = edge_index[0]
    dst = edge_index[1]
    pad_e = ep - e
    dstp = jnp.concatenate([dst, jnp.full((pad_e,), dummy, jnp.int32)])
    srcp = jnp.concatenate([src, jnp.full((pad_e,), dummy, jnp.int32)])
    x_pad = jnp.pad(x, ((0, npad - n), (0, 0)))

    wcat = jnp.concatenate([W1[:c], W1[c:], W_root], axis=1)

    a0, a1, b0, b1v, xr = _node_linear(x_pad, wcat, b1, npad, 1024)

    rpt = npad // _NS
    z = jnp.zeros((rpt, 128), jnp.float32)
    s_out = _sc_edge_kernel(a0, a1, b0, b1v, dstp, srcp, z, npad, ep)

    p, stats = _epilogue_a(s_out, xr[:n], W2, b2, n, npad, 1000)
    out = _epilogue_b(p, stats, gamma, beta, n, 1000)
    return (out, edge_index, edge_attr, batch)


# speedup vs baseline: 2.2083x; 1.1256x over previous
import functools

import jax
import jax.numpy as jnp
from jax import lax
from jax.experimental import pallas as pl
from jax.experimental.pallas import tpu as pltpu
from jax.experimental.pallas import tpu_sc as plsc

_NC = 2
_NS = 16
_LANES = 16

_K = 80
_DEGW = 16


def _linear_body(xb, wc, b1r, a0, a1, b0, b1o, xr):
    y = jnp.dot(xb[...], wc[...], preferred_element_type=jnp.float32)
    a0[...] = y[:, 0:128] + b1r[:, 0:128]
    a1[...] = y[:, 128:256] + b1r[:, 128:256]
    b0[...] = y[:, 256:384]
    b1o[...] = y[:, 384:512]
    xr[...] = y[:, 512:768]


def _node_linear(x_pad, wcat, b1, npad, rb):
    grid = npad // rb
    f32 = jnp.float32
    outs = [
        jax.ShapeDtypeStruct((npad, 128), f32),
        jax.ShapeDtypeStruct((npad, 128), f32),
        jax.ShapeDtypeStruct((npad, 128), f32),
        jax.ShapeDtypeStruct((npad, 128), f32),
        jax.ShapeDtypeStruct((npad, 256), f32),
    ]
    half_spec = pl.BlockSpec((rb, 128), lambda i: (i, 0))
    full_spec = pl.BlockSpec((rb, 256), lambda i: (i, 0))
    return pl.pallas_call(
        _linear_body,
        grid=(grid,),
        in_specs=[
            pl.BlockSpec((rb, 256), lambda i: (i, 0)),
            pl.BlockSpec((256, 768), lambda i: (0, 0)),
            pl.BlockSpec((1, 256), lambda i: (0, 0)),
        ],
        out_specs=[half_spec, half_spec, half_spec, half_spec, full_spec],
        out_shape=outs,
    )(x_pad, wcat, b1.reshape(1, 256))


def _sc_edge_kernel(a0, a1, b0, b1, dstp, srcp, z, npad, ep):
    ec = ep // (_NC * _NS)
    nb = ec // _K
    rpt = npad // _NS
    f32 = jnp.float32

    def body(a0_h, a1_h, b0_h, b1_h, dst_h, src_h, z_h,
             s_out,
             dstw0, srcw0, bufa0, bufb0,
             dstw1, srcw1, bufa1, bufb1,
             s_sh,
             sa0, sb0, sa1, sb1, sc0, sc1):
        cid = lax.axis_index("c")
        sid = lax.axis_index("s")
        gid = cid * _NS + sid
        base = gid * ec
        sets = ((dstw0, srcw0, bufa0, bufb0, sa0, sb0, sc0),
                (dstw1, srcw1, bufa1, bufb1, sa1, sb1, sc1))

        for h in range(3):
            ah = a0_h if h == 0 else a1_h
            bh = b0_h if h == 0 else b1_h
            pltpu.sync_copy(z_h, s_sh.at[pl.ds(sid * rpt, rpt)])
            plsc.subcore_barrier()

            if h < 2:
                def fire(b, p):
                    dw, sw, ba, bb, sa, sb, _ = sets[p]
                    off = base + b * _K
                    pltpu.sync_copy(dst_h.at[pl.ds(off, _K)], dw)
                    pltpu.sync_copy(src_h.at[pl.ds(off, _K)], sw)
                    pltpu.async_copy(ah.at[dw], ba, sa)
                    pltpu.async_copy(bh.at[sw], bb, sb)

                fire(0, 0)

                def pair(g, c):
                    for p in range(2):
                        q = 1 - p
                        dw, sw, ba, bb, sa, sb, sc = sets[p]
                        dwq, swq, baq, bbq, saq, sbq, scq = sets[q]
                        b = 2 * g + p
                        pltpu.make_async_copy(ah.at[dw], ba, sa).wait()
                        pltpu.make_async_copy(bh.at[sw], bb, sb).wait()

                        @pl.when(b > 0)
                        def _drain_prev():
                            pltpu.make_async_copy(
                                baq, s_sh.at[dwq], scq).wait()

                        @pl.when(b + 1 < nb)
                        def _stage_next():
                            fire(b + 1, q)

                        @plsc.parallel_loop(0, _K, step=1, unroll=4)
                        def row(r):
                            for cc in range(128 // _LANES):
                                av = ba[r, pl.ds(cc * _LANES, _LANES)]
                                bv = bb[r, pl.ds(cc * _LANES, _LANES)]
                                ba[r, pl.ds(cc * _LANES, _LANES)] = (
                                    jnp.maximum(av + bv, 0.0))

                        pltpu.async_copy(ba, s_sh.at[dw], sc, add=True)
                    return c
                lax.fori_loop(0, nb // 2, pair, 0)
                pltpu.make_async_copy(bufa1, s_sh.at[dstw1], sc1).wait()
            else:
                bufa = bufa0

                def fill_ones(r, c):
                    for cc in range(128 // _LANES):
                        bufa[r, pl.ds(cc * _LANES, _LANES)] = jnp.full(
                            (_LANES,), 1.0, f32)
                    return c
                lax.fori_loop(0, _K, fill_ones, 0)

                def dpair(g, c):
                    for p in range(2):
                        dw = sets[p][0]
                        sc = sets[p][6]
                        b = 2 * g + p

                        @pl.when(b > 1)
                        def _drain_same():
                            pltpu.make_async_copy(
                                bufa, s_sh.at[dw], sc).wait()

                        off = base + b * _K
                        pltpu.sync_copy(dst_h.at[pl.ds(off, _K)], dw)
                        pltpu.async_copy(bufa, s_sh.at[dw], sc, add=True)
                    return c
                lax.fori_loop(0, nb // 2, dpair, 0)
                pltpu.make_async_copy(bufa, s_sh.at[dstw0], sc0).wait()
                pltpu.make_async_copy(bufa, s_sh.at[dstw1], sc1).wait()

            plsc.subcore_barrier()
            pltpu.sync_copy(s_sh.at[pl.ds(sid * rpt, rpt)],
                            s_out.at[cid, h, pl.ds(sid * rpt, rpt)])
            plsc.subcore_barrier()

    mesh = plsc.VectorSubcoreMesh(core_axis_name="c", subcore_axis_name="s")
    kern = pl.kernel(
        body,
        out_type=jax.ShapeDtypeStruct((_NC, 3, npad, 128), f32),
        mesh=mesh,
        scratch_types=[
            pltpu.VMEM((_K,), jnp.int32),
            pltpu.VMEM((_K,), jnp.int32),
            pltpu.VMEM((_K, 128), f32),
            pltpu.VMEM((_K, 128), f32),
            pltpu.VMEM((_K,), jnp.int32),
            pltpu.VMEM((_K,), jnp.int32),
            pltpu.VMEM((_K, 128), f32),
            pltpu.VMEM((_K, 128), f32),
            pltpu.VMEM_SHARED((npad, 128), f32),
            pltpu.SemaphoreType.DMA,
            pltpu.SemaphoreType.DMA,
            pltpu.SemaphoreType.DMA,
            pltpu.SemaphoreType.DMA,
            pltpu.SemaphoreType.DMA,
            pltpu.SemaphoreType.DMA,
        ],
    )
    return kern(a0, a1, b0, b1, dstp, srcp, z)


def _agg_body(nblk, s_ref, xr_ref, w2_ref, b2r, p_ref, stats_ref,
              acc_ref):
    i = pl.program_id(0)
    sh0 = s_ref[0, 0] + s_ref[1, 0]
    sh1 = s_ref[0, 1] + s_ref[1, 1]
    p = jnp.dot(sh0, w2_ref[0:128, :], preferred_element_type=jnp.float32)
    p += jnp.dot(sh1, w2_ref[128:256, :], preferred_element_type=jnp.float32)
    dcol = s_ref[0, 2, :, 0:1] + s_ref[1, 2, :, 0:1]
    p = p + dcol * b2r[...] + xr_ref[...]
    p_ref[...] = p

    @pl.when(i == 0)
    def _init():
        acc_ref[...] = jnp.zeros_like(acc_ref)

    acc_ref[0:1, :] += jnp.sum(p, axis=0, keepdims=True)
    acc_ref[1:2, :] += jnp.sum(p * p, axis=0, keepdims=True)
    stats_ref[...] = acc_ref[...]


def _epilogue_a(s_out, xr, w2, b2, n, npad, rb):
    grid = n // rb
    f32 = jnp.float32
    return pl.pallas_call(
        functools.partial(_agg_body, grid),
        grid=(grid,),
        in_specs=[
            pl.BlockSpec((_NC, 3, rb, 128), lambda i: (0, 0, i, 0)),
            pl.BlockSpec((rb, 256), lambda i: (i, 0)),
            pl.BlockSpec((256, 256), lambda i: (0, 0)),
            pl.BlockSpec((1, 256), lambda i: (0, 0)),
        ],
        out_specs=[
            pl.BlockSpec((rb, 256), lambda i: (i, 0)),
            pl.BlockSpec((8, 256), lambda i: (0, 0)),
        ],
        out_shape=[
            jax.ShapeDtypeStruct((n, 256), f32),
            jax.ShapeDtypeStruct((8, 256), f32),
        ],
        scratch_shapes=[pltpu.VMEM((8, 256), f32)],
    )(s_out, xr, w2, b2.reshape(1, 256))


def _bn_body(n, p_ref, stats_ref, g_ref, be_ref, o_ref):
    inv_n = 1.0 / n
    mean = stats_ref[0:1, :] * inv_n
    ex2 = stats_ref[1:2, :] * inv_n
    var = ex2 - mean * mean
    rstd = lax.rsqrt(var + 1e-5)
    o_ref[...] = jnp.maximum(
        (p_ref[...] - mean) * rstd * g_ref[...] + be_ref[...], 0.0)


def _epilogue_b(p, stats, gamma, beta, n, rb):
    return pl.pallas_call(
        functools.partial(_bn_body, float(n)),
        grid=(n // rb,),
        in_specs=[
            pl.BlockSpec((rb, 256), lambda i: (i, 0)),
            pl.BlockSpec((8, 256), lambda i: (0, 0)),
            pl.BlockSpec((1, 256), lambda i: (0, 0)),
            pl.BlockSpec((1, 256), lambda i: (0, 0)),
        ],
        out_specs=pl.BlockSpec((rb, 256), lambda i: (i, 0)),
        out_shape=jax.ShapeDtypeStruct((n, 256), jnp.float32),
    )(p, stats, gamma.reshape(1, 256), beta.reshape(1, 256))


def kernel(x, edge_index, edge_attr, batch, W1, b1, W2, b2, W_root, gamma,
           beta):
    n, c = x.shape
    e = edge_index.shape[1]
    npad = 10240
    unit = 32 * 2 * _K
    ep = ((e + unit - 1) // unit) * unit
    dummy = n

    src = edge_index[0]
    dst = edge_index[1]
    pad_e = ep - e
    dstp = jnp.concatenate([dst, jnp.full((pad_e,), dummy, jnp.int32)])
    srcp = jnp.concatenate([src, jnp.full((pad_e,), dummy, jnp.int32)])
    x_pad = jnp.pad(x, ((0, npad - n), (0, 0)))

    wcat = jnp.concatenate([W1[:c], W1[c:], W_root], axis=1)

    a0, a1, b0, b1v, xr = _node_linear(x_pad, wcat, b1, npad, 1024)

    rpt = npad // _NS
    z = jnp.zeros((rpt, 128), jnp.float32)
    s_out = _sc_edge_kernel(a0, a1, b0, b1v, dstp, srcp, z, npad, ep)

    p, stats = _epilogue_a(s_out, xr[:n], W2, b2, n, npad, 1000)
    out = _epilogue_b(p, stats, gamma, beta, n, 1000)
    return (out, edge_index, edge_attr, batch)
